# Initial kernel scaffold; baseline (speedup 1.0000x reference)
import jax
import jax.numpy as jnp
from jax.experimental import pallas as pl


def kernel(x, cw1, cb1, cw2, cb2, cw3, cb3, an_logs, an_b):
    raise NotImplementedError("write your pallas kernel here")


# speedup vs baseline: 1.0286x; 1.0286x over previous
import jax
import jax.numpy as jnp
from jax.experimental import pallas as pl
from jax.experimental.pallas import tpu as pltpu

_N_COUPLING = 4
_D = 2
_H = 512
_BB = 2048


def _flow_kernel(x_ref, cw1_ref, cb1_ref, cw2_ref, cb2_ref, cw3_ref,
                 cb3_ref, ansc_ref, anb_ref, ansum_ref, out_ref, ld_ref):
    a = x_ref[:, 0:1]
    b = x_ref[:, 1:2]
    ld = jnp.zeros((_BB, 1), jnp.float32)

    for i in range(_N_COUPLING):
        idx = i % 2
        nm = 1 - idx
        cond = a if idx == 0 else b
        other = b if idx == 0 else a

        w1row = cw1_ref[i, idx:idx + 1, :]
        b1row = cb1_ref[i:i + 1, :]
        h1 = jnp.maximum(cond.astype(jnp.bfloat16) * w1row + b1row, 0)
        acc = jnp.dot(h1, cw2_ref[i], preferred_element_type=jnp.float32)
        h2 = jnp.maximum(acc.astype(jnp.bfloat16) + cb2_ref[i:i + 1, :], 0)
        st = jnp.dot(h2, cw3_ref[i], preferred_element_type=jnp.float32)

        log_s = jnp.tanh(st[:, nm:nm + 1] + cb3_ref[4 * i + nm])
        t = st[:, 2 + nm:3 + nm] + cb3_ref[4 * i + 2 + nm]
        other = other * jnp.exp(log_s) + t
        ld = ld + log_s

        new0 = cond if idx == 0 else other
        new1 = other if idx == 0 else cond
        a = new0 * ansc_ref[2 * i + 0] + anb_ref[2 * i + 0]
        b = new1 * ansc_ref[2 * i + 1] + anb_ref[2 * i + 1]

    ld = ld + ansum_ref[0]

    for z, col in ((a, 0), (b, 1)):
        u = jnp.abs(z)
        ld = ld - (2.0 * jnp.log1p(jnp.exp(-u)) + u)
        out_ref[:, col:col + 1] = jax.nn.sigmoid(z)
    ld_ref[...] = ld


def kernel(x, cw1, cb1, cw2, cb2, cw3, cb3, an_logs, an_b):
    B = x.shape[0]
    cw1b = cw1.astype(jnp.bfloat16)
    cb1b = cb1.astype(jnp.bfloat16)
    cw2b = cw2.astype(jnp.bfloat16)
    cb2b = cb2.astype(jnp.bfloat16)
    cw3b = cw3.astype(jnp.bfloat16)
    cb3f = cb3.reshape(-1)
    ansc = jnp.exp(an_logs).reshape(-1)
    anbf = an_b.reshape(-1)
    ansum = jnp.sum(an_logs).reshape(1)

    out, ld = pl.pallas_call(
        _flow_kernel,
        out_shape=(jax.ShapeDtypeStruct((B, _D), jnp.float32),
                   jax.ShapeDtypeStruct((B, 1), jnp.float32)),
        grid=(B // _BB,),
        in_specs=[
            pl.BlockSpec((_BB, _D), lambda i: (i, 0)),
            pl.BlockSpec((_N_COUPLING, _D, _H), lambda i: (0, 0, 0)),
            pl.BlockSpec((_N_COUPLING, _H), lambda i: (0, 0)),
            pl.BlockSpec((_N_COUPLING, _H, _H), lambda i: (0, 0, 0)),
            pl.BlockSpec((_N_COUPLING, _H), lambda i: (0, 0)),
            pl.BlockSpec((_N_COUPLING, _H, 2 * _D), lambda i: (0, 0, 0)),
            pl.BlockSpec(memory_space=pltpu.SMEM),
            pl.BlockSpec(memory_space=pltpu.SMEM),
            pl.BlockSpec(memory_space=pltpu.SMEM),
            pl.BlockSpec(memory_space=pltpu.SMEM),
        ],
        out_specs=(pl.BlockSpec((_BB, _D), lambda i: (i, 0)),
                   pl.BlockSpec((_BB, 1), lambda i: (i, 0))),
        compiler_params=pltpu.CompilerParams(
            dimension_semantics=("parallel",),
            vmem_limit_bytes=48 * 1024 * 1024,
        ),
        name="realnvp2d_fused",
    )(x, cw1b, cb1b, cw2b, cb2b, cw3b, cb3f, ansc, anbf, ansum)
    return out, ld.reshape(B)
